# baseline (speedup 1.0000x reference)
import functools

import jax
import jax.numpy as jnp
from jax import lax
from jax.experimental import pallas as pl
from jax.experimental.pallas import tpu as pltpu
from jax.experimental.pallas import tpu_sc as plsc

NC = 2
NS = 16
NW = NC * NS

R = 16


def _make_gather(vocab, dim, n):
    assert n % NW == 0
    b_per_w = n // NW
    assert b_per_w % R == 0
    n_groups = b_per_w // R
    assert n_groups >= 2

    mesh = plsc.VectorSubcoreMesh(core_axis_name="c", subcore_axis_name="s")

    @functools.partial(
        pl.kernel,
        out_type=jax.ShapeDtypeStruct((n, dim), jnp.float32),
        mesh=mesh,
        scratch_types=[
            pltpu.VMEM((b_per_w,), jnp.int32),
            pltpu.VMEM_SHARED((NS, 2, R, dim), jnp.float32),
            [pltpu.SemaphoreType.DMA for _ in range(2)],
            [pltpu.SemaphoreType.DMA for _ in range(2)],
        ],
    )
    def gather(table_hbm, idx_hbm, out_hbm, idx_v, shared, gsems, ssems):
        cid = lax.axis_index("c")
        sid = lax.axis_index("s")
        wid = sid * NC + cid
        base = wid * b_per_w
        pltpu.sync_copy(idx_hbm.at[pl.ds(base, b_per_w)], idx_v)

        def issue_group(g, slot):
            vec = idx_v[pl.ds(g * R, R)]
            for j in range(R):
                row = vec[j]
                pltpu.async_copy(
                    table_hbm.at[pl.ds(row, 1)],
                    shared.at[sid, slot, pl.ds(j, 1)],
                    gsems[slot],
                )

        def wait_group(slot):
            pltpu.make_async_copy(
                table_hbm.at[pl.ds(0, R)], shared.at[sid, slot], gsems[slot]
            ).wait()

        issue_group(0, 0)

        def body(g, _):
            nxt = g + 1
            for slot in range(2):
                @pl.when(lax.rem(g, 2) == slot)
                def _():
                    other = 1 - slot
                    @pl.when(nxt < n_groups)
                    def _():
                        @pl.when(nxt >= 2)
                        def _():
                            pltpu.make_async_copy(
                                shared.at[sid, other],
                                out_hbm.at[pl.ds(base, R)],
                                ssems[other],
                            ).wait()
                        issue_group(nxt, other)
                    wait_group(slot)
                    pltpu.async_copy(
                        shared.at[sid, slot],
                        out_hbm.at[pl.ds(base + g * R, R)],
                        ssems[slot],
                    )
            return 0

        lax.fori_loop(0, n_groups, body, 0)

        for slot in range(2):
            pltpu.make_async_copy(
                shared.at[sid, slot], out_hbm.at[pl.ds(base, R)], ssems[slot]
            ).wait()

    return gather


K = 16
N_TC = 2560


def _tc_gather(vocab, dim, n):
    assert n % K == 0

    def body(idx_ref, *refs):
        in_refs = refs[:K]
        out_ref = refs[K]
        for j in range(K):
            out_ref[j] = in_refs[j][0]

    grid_spec = pltpu.PrefetchScalarGridSpec(
        num_scalar_prefetch=1,
        grid=(n // K,),
        in_specs=[
            pl.BlockSpec((1, 1, dim), lambda i, idx_ref, j=j: (idx_ref[K * i + j], 0, 0))
            for j in range(K)
        ],
        out_specs=pl.BlockSpec((K, 1, dim), lambda i, idx_ref: (i, 0, 0)),
    )
    return pl.pallas_call(
        body,
        grid_spec=grid_spec,
        out_shape=jax.ShapeDtypeStruct((n, 1, dim), jnp.float32),
    )


def kernel(input_ids, weight):
    b, s = input_ids.shape
    vocab, dim = weight.shape
    idx = input_ids.reshape(-1).astype(jnp.int32)
    n = idx.shape[0]
    n_sc = n - N_TC
    sc_out = _make_gather(vocab, dim, n_sc)(weight, idx[:n_sc])
    w3 = weight.reshape(vocab, 1, dim)
    tc_out = _tc_gather(vocab, dim, N_TC)(idx[n_sc:], *([w3] * K))
    out = jnp.concatenate([sc_out, tc_out.reshape(N_TC, dim)], axis=0)
    return out.reshape(b, s, dim)


# speedup vs baseline: 7.1327x; 7.1327x over previous
import functools

import jax
import jax.numpy as jnp
from jax import lax
from jax.experimental import pallas as pl
from jax.experimental.pallas import tpu as pltpu
from jax.experimental.pallas import tpu_sc as plsc

NC = 2
NS = 16
NW = NC * NS

R = 8
NSLOT = 4
LA = 2


def _make_gather(vocab, dim, n):
    assert n % NW == 0
    b_per_w = n // NW
    assert b_per_w % R == 0
    n_groups = b_per_w // R
    assert n_groups >= NSLOT

    mesh = plsc.VectorSubcoreMesh(core_axis_name="c", subcore_axis_name="s")

    @functools.partial(
        pl.kernel,
        out_type=jax.ShapeDtypeStruct((n, dim), jnp.float32),
        mesh=mesh,
        scratch_types=[
            pltpu.VMEM((b_per_w + 16,), jnp.int32),
            pltpu.VMEM_SHARED((NS, NSLOT, R, dim), jnp.float32),
            [pltpu.SemaphoreType.DMA for _ in range(NSLOT)],
            [pltpu.SemaphoreType.DMA for _ in range(NSLOT)],
        ],
    )
    def gather(table_hbm, idx_hbm, out_hbm, idx_v, shared, gsems, ssems):
        cid = lax.axis_index("c")
        sid = lax.axis_index("s")
        wid = sid * NC + cid
        base = wid * b_per_w
        pltpu.sync_copy(idx_hbm.at[pl.ds(base, b_per_w)],
                        idx_v.at[pl.ds(0, b_per_w)])

        def issue_group(g, slot):
            vec = idx_v[pl.ds(g * R, 16)]
            for j in range(R):
                row = vec[j]
                pltpu.async_copy(
                    table_hbm.at[pl.ds(row, 1)],
                    shared.at[sid, slot, pl.ds(j, 1)],
                    gsems[slot],
                )

        for g in range(LA):
            issue_group(g, g % NSLOT)

        def body(g, _):
            nxt = g + LA
            for slot in range(NSLOT):
                @pl.when(lax.rem(nxt, NSLOT) == slot)
                def _():
                    @pl.when(nxt < n_groups)
                    def _():
                        @pl.when(nxt >= NSLOT)
                        def _():
                            pltpu.make_async_copy(
                                shared.at[sid, slot],
                                out_hbm.at[pl.ds(base, R)],
                                ssems[slot],
                            ).wait()
                        issue_group(nxt, slot)
            for slot in range(NSLOT):
                @pl.when(lax.rem(g, NSLOT) == slot)
                def _():
                    pltpu.make_async_copy(
                        table_hbm.at[pl.ds(0, R)], shared.at[sid, slot],
                        gsems[slot],
                    ).wait()
                    pltpu.async_copy(
                        shared.at[sid, slot],
                        out_hbm.at[pl.ds(base + g * R, R)],
                        ssems[slot],
                    )
            return 0

        lax.fori_loop(0, n_groups, body, 0)

        for slot in range(NSLOT):
            pltpu.make_async_copy(
                shared.at[sid, slot], out_hbm.at[pl.ds(base, R)], ssems[slot]
            ).wait()

    return gather


def kernel(input_ids, weight):
    b, s = input_ids.shape
    vocab, dim = weight.shape
    idx = input_ids.reshape(-1).astype(jnp.int32)
    out = _make_gather(vocab, dim, idx.shape[0])(weight, idx)
    return out.reshape(b, s, dim)
